# Initial kernel scaffold; baseline (speedup 1.0000x reference)
import jax
import jax.numpy as jnp
from jax.experimental import pallas as pl


def kernel(x, edge_index, W_c1, b_c1, W_c2, b_c2, W_c3, b_c3, W_l1, b_l1, W_l2, b_l2):
    raise NotImplementedError("write your pallas kernel here")


# speedup vs baseline: 488.9259x; 488.9259x over previous
import functools

import jax
import jax.numpy as jnp
from jax import lax
from jax.experimental import pallas as pl
from jax.experimental.pallas import tpu as pltpu
from jax.experimental.pallas import tpu_sc as plsc

_N = 8
_E = 1048576
_NC = 2
_NS = 16
_NW = _NC * _NS
_L = 16
_CH = _E // _NW
_NBINS = _N * _N


def _sc_edge_histogram(edge_flat):
    mesh = plsc.VectorSubcoreMesh(core_axis_name="c", subcore_axis_name="s")

    @functools.partial(
        pl.kernel,
        out_type=jax.ShapeDtypeStruct((_NW * _NBINS,), jnp.float32),
        mesh=mesh,
        scratch_types=[
            pltpu.VMEM((_CH,), jnp.int32),
            pltpu.VMEM((_CH,), jnp.int32),
            pltpu.VMEM((_L * _NBINS,), jnp.float32),
            pltpu.VMEM((_NBINS,), jnp.float32),
            pltpu.SemaphoreType.DMA,
            pltpu.SemaphoreType.DMA,
        ],
        compiler_params=pltpu.CompilerParams(needs_layout_passes=False),
    )
    def hist(edge_hbm, out_hbm, rows_v, cols_v, acc_v, cnt_v, sem_r, sem_c):
        wid = lax.axis_index("s") * _NC + lax.axis_index("c")
        base = wid * _CH
        cp_r = pltpu.async_copy(edge_hbm.at[pl.ds(base, _CH)], rows_v, sem_r)
        cp_c = pltpu.async_copy(edge_hbm.at[pl.ds(_E + base, _CH)], cols_v,
                                sem_c)

        zeros = jnp.zeros((_L,), jnp.float32)
        for j in range(_NBINS):
            acc_v[pl.ds(j * _L, _L)] = zeros

        lane_base = lax.iota(jnp.int32, _L) * _NBINS
        ones = jnp.ones((_L,), jnp.float32)
        cp_r.wait()
        cp_c.wait()

        def body(i, carry):
            r = rows_v[pl.ds(i * _L, _L)]
            c = cols_v[pl.ds(i * _L, _L)]
            addr = lane_base + r * _N + c
            plsc.addupdate_scatter(acc_v, [addr], ones)
            return carry

        lax.fori_loop(0, _CH // _L, body, 0)

        for kk in range(_NBINS // _L):
            s = acc_v[pl.ds(kk * _L, _L)]
            for l in range(1, _L):
                s = s + acc_v[pl.ds(l * _NBINS + kk * _L, _L)]
            cnt_v[pl.ds(kk * _L, _L)] = s

        pltpu.sync_copy(cnt_v, out_hbm.at[pl.ds(wid * _NBINS, _NBINS)])

    return hist(edge_flat)


def _tc_head(parts, x, W1, b1, W2, b2, W3, b3, Wl1t, bl1, Wl2, bl2):
    def body(p_ref, x_ref, w1_ref, b1_ref, w2_ref, b2_ref, w3_ref, b3_ref,
             wl1_ref, bl1_ref, wl2_ref, bl2_ref, o_ref):
        C = jnp.sum(p_ref[...], axis=0)
        ii = lax.broadcasted_iota(jnp.int32, (_N, _N), 0)
        jj = lax.broadcasted_iota(jnp.int32, (_N, _N), 1)
        C = C + (ii == jj).astype(jnp.float32)
        deg = jnp.sum(C, axis=0, keepdims=True)
        dis = lax.rsqrt(deg)
        Dm = (ii == jj).astype(jnp.float32) * dis

        def dot(a, b, dims):
            return lax.dot_general(a, b, (dims, ((), ())),
                                   preferred_element_type=jnp.float32)

        A = dot(Dm, dot(C, Dm, (((0,), (0,)))), (((1,), (0,))))

        h = dot(x_ref[...], w1_ref[...], (((1,), (1,))))
        h = jnp.maximum(dot(A, h, (((1,), (0,)))) + b1_ref[...], 0.0)
        h = dot(h, w2_ref[...], (((1,), (1,))))
        h = jnp.maximum(dot(A, h, (((1,), (0,)))) + b2_ref[...], 0.0)
        h = dot(h, w3_ref[...], (((1,), (1,))))
        h = jnp.maximum(dot(A, h, (((1,), (0,)))) + b3_ref[...], 0.0)

        lin1 = bl1_ref[...]
        for n in range(_N):
            lin1 = lin1 + dot(h[n:n + 1, :],
                              wl1_ref[n * 16:(n + 1) * 16, :],
                              (((1,), (0,))))
        q = jnp.maximum(lin1, 0.0)
        o_ref[...] = dot(q, wl2_ref[...], (((1,), (1,)))) + bl2_ref[...]

    return pl.pallas_call(
        body,
        out_shape=jax.ShapeDtypeStruct((1, 16), jnp.float32),
    )(parts, x, W1, b1, W2, b2, W3, b3, Wl1t, bl1, Wl2, bl2)


def kernel(x, edge_index, W_c1, b_c1, W_c2, b_c2, W_c3, b_c3,
           W_l1, b_l1, W_l2, b_l2):
    parts = _sc_edge_histogram(edge_index.reshape(-1))
    return _tc_head(
        parts.reshape(_NW, _N, _N), x,
        W_c1, b_c1.reshape(1, -1),
        W_c2, b_c2.reshape(1, -1),
        W_c3, b_c3.reshape(1, -1),
        W_l1.T, b_l1.reshape(1, -1),
        W_l2, b_l2.reshape(1, -1),
    )


# speedup vs baseline: 492.1458x; 1.0066x over previous
import functools

import jax
import jax.numpy as jnp
from jax import lax
from jax.experimental import pallas as pl
from jax.experimental.pallas import tpu as pltpu
from jax.experimental.pallas import tpu_sc as plsc

_N = 8
_E = 1048576
_NC = 2
_NS = 16
_NW = _NC * _NS
_L = 16
_CH = _E // _NW
_NBINS = _N * _N
_UNROLL = 8


def _sc_edge_histogram(edge_flat):
    mesh = plsc.VectorSubcoreMesh(core_axis_name="c", subcore_axis_name="s")

    @functools.partial(
        pl.kernel,
        out_type=jax.ShapeDtypeStruct((_NW * _NBINS,), jnp.float32),
        mesh=mesh,
        scratch_types=[
            pltpu.VMEM((_CH,), jnp.int32),
            pltpu.VMEM((_CH,), jnp.int32),
            pltpu.VMEM((_L * _NBINS,), jnp.float32),
            pltpu.VMEM((_NBINS,), jnp.float32),
            pltpu.SemaphoreType.DMA,
            pltpu.SemaphoreType.DMA,
        ],
        compiler_params=pltpu.CompilerParams(needs_layout_passes=False),
    )
    def hist(edge_hbm, out_hbm, rows_v, cols_v, acc_v, cnt_v, sem_r, sem_c):
        wid = lax.axis_index("s") * _NC + lax.axis_index("c")
        base = wid * _CH
        cp_r = pltpu.async_copy(edge_hbm.at[pl.ds(base, _CH)], rows_v, sem_r)
        cp_c = pltpu.async_copy(edge_hbm.at[pl.ds(_E + base, _CH)], cols_v,
                                sem_c)

        zeros = jnp.zeros((_L,), jnp.float32)
        for j in range(_NBINS):
            acc_v[pl.ds(j * _L, _L)] = zeros

        lane_base = lax.iota(jnp.int32, _L) * _NBINS
        ones = jnp.ones((_L,), jnp.float32)
        cp_r.wait()
        cp_c.wait()

        def body(i, carry):
            base_i = i * (_L * _UNROLL)
            for u in range(_UNROLL):
                r = rows_v[pl.ds(base_i + u * _L, _L)]
                c = cols_v[pl.ds(base_i + u * _L, _L)]
                addr = lane_base + r * _N + c
                plsc.addupdate_scatter(acc_v, [addr], ones)
            return carry

        lax.fori_loop(0, _CH // (_L * _UNROLL), body, 0)

        for kk in range(_NBINS // _L):
            s = acc_v[pl.ds(kk * _L, _L)]
            for l in range(1, _L):
                s = s + acc_v[pl.ds(l * _NBINS + kk * _L, _L)]
            cnt_v[pl.ds(kk * _L, _L)] = s

        pltpu.sync_copy(cnt_v, out_hbm.at[pl.ds(wid * _NBINS, _NBINS)])

    return hist(edge_flat)


def _tc_head(parts, x, W1, b1, W2, b2, W3, b3, Wl1t, bl1, Wl2, bl2):
    def body(p_ref, x_ref, w1_ref, b1_ref, w2_ref, b2_ref, w3_ref, b3_ref,
             wl1_ref, bl1_ref, wl2_ref, bl2_ref, o_ref):
        C = jnp.sum(p_ref[...], axis=0)
        ii = lax.broadcasted_iota(jnp.int32, (_N, _N), 0)
        jj = lax.broadcasted_iota(jnp.int32, (_N, _N), 1)
        C = C + (ii == jj).astype(jnp.float32)
        deg = jnp.sum(C, axis=0, keepdims=True)
        dis = lax.rsqrt(deg)
        Dm = (ii == jj).astype(jnp.float32) * dis

        def dot(a, b, dims):
            return lax.dot_general(a, b, (dims, ((), ())),
                                   preferred_element_type=jnp.float32)

        A = dot(Dm, dot(C, Dm, (((0,), (0,)))), (((1,), (0,))))

        h = dot(x_ref[...], w1_ref[...], (((1,), (1,))))
        h = jnp.maximum(dot(A, h, (((1,), (0,)))) + b1_ref[...], 0.0)
        h = dot(h, w2_ref[...], (((1,), (1,))))
        h = jnp.maximum(dot(A, h, (((1,), (0,)))) + b2_ref[...], 0.0)
        h = dot(h, w3_ref[...], (((1,), (1,))))
        h = jnp.maximum(dot(A, h, (((1,), (0,)))) + b3_ref[...], 0.0)

        lin1 = bl1_ref[...]
        for n in range(_N):
            lin1 = lin1 + dot(h[n:n + 1, :],
                              wl1_ref[n * 16:(n + 1) * 16, :],
                              (((1,), (0,))))
        q = jnp.maximum(lin1, 0.0)
        o_ref[...] = dot(q, wl2_ref[...], (((1,), (1,)))) + bl2_ref[...]

    return pl.pallas_call(
        body,
        out_shape=jax.ShapeDtypeStruct((1, 16), jnp.float32),
    )(parts, x, W1, b1, W2, b2, W3, b3, Wl1t, bl1, Wl2, bl2)


def kernel(x, edge_index, W_c1, b_c1, W_c2, b_c2, W_c3, b_c3,
           W_l1, b_l1, W_l2, b_l2):
    parts = _sc_edge_histogram(edge_index.reshape(-1))
    return _tc_head(
        parts.reshape(_NW, _N, _N), x,
        W_c1, b_c1.reshape(1, -1),
        W_c2, b_c2.reshape(1, -1),
        W_c3, b_c3.reshape(1, -1),
        W_l1.T, b_l1.reshape(1, -1),
        W_l2, b_l2.reshape(1, -1),
    )


# speedup vs baseline: 630.5476x; 1.2812x over previous
import functools

import jax
import jax.numpy as jnp
from jax import lax
from jax.experimental import pallas as pl
from jax.experimental.pallas import tpu as pltpu
from jax.experimental.pallas import tpu_sc as plsc

_N = 8
_E = 1048576
_NC = 2
_NS = 16
_NW = _NC * _NS
_L = 16
_CH = _E // _NW
_NBINS = _N * _N
_UNROLL = 8


def _sc_edge_histogram(edge_index_2d):
    mesh = plsc.VectorSubcoreMesh(core_axis_name="c", subcore_axis_name="s")

    @functools.partial(
        pl.kernel,
        out_type=jax.ShapeDtypeStruct((_NW * _NBINS,), jnp.float32),
        mesh=mesh,
        scratch_types=[
            pltpu.VMEM((_CH,), jnp.int32),
            pltpu.VMEM((_CH,), jnp.int32),
            pltpu.VMEM((_L * _NBINS,), jnp.float32),
            pltpu.VMEM((_NBINS,), jnp.float32),
            pltpu.SemaphoreType.DMA,
            pltpu.SemaphoreType.DMA,
        ],
        compiler_params=pltpu.CompilerParams(needs_layout_passes=False),
    )
    def hist(edge_hbm, out_hbm, rows_v, cols_v, acc_v, cnt_v, sem_r, sem_c):
        wid = lax.axis_index("s") * _NC + lax.axis_index("c")
        base = wid * _CH
        cp_r = pltpu.async_copy(edge_hbm.at[0, pl.ds(base, _CH)], rows_v,
                                sem_r)
        cp_c = pltpu.async_copy(edge_hbm.at[1, pl.ds(base, _CH)], cols_v,
                                sem_c)

        zeros = jnp.zeros((_L,), jnp.float32)
        for j in range(_NBINS):
            acc_v[pl.ds(j * _L, _L)] = zeros

        lane_base = lax.iota(jnp.int32, _L) * _NBINS
        ones = jnp.ones((_L,), jnp.float32)
        cp_r.wait()
        cp_c.wait()

        def body(i, carry):
            base_i = i * (_L * _UNROLL)
            for u in range(_UNROLL):
                r = rows_v[pl.ds(base_i + u * _L, _L)]
                c = cols_v[pl.ds(base_i + u * _L, _L)]
                addr = lane_base + r * _N + c
                plsc.addupdate_scatter(acc_v, [addr], ones)
            return carry

        lax.fori_loop(0, _CH // (_L * _UNROLL), body, 0)

        for kk in range(_NBINS // _L):
            s = acc_v[pl.ds(kk * _L, _L)]
            for l in range(1, _L):
                s = s + acc_v[pl.ds(l * _NBINS + kk * _L, _L)]
            cnt_v[pl.ds(kk * _L, _L)] = s

        pltpu.sync_copy(cnt_v, out_hbm.at[pl.ds(wid * _NBINS, _NBINS)])

    return hist(edge_index_2d)


def _tc_head(parts, x, W1, b1, W2, b2, W3, b3, Wl1t, bl1, Wl2, bl2):
    def body(p_ref, x_ref, w1_ref, b1_ref, w2_ref, b2_ref, w3_ref, b3_ref,
             wl1_ref, bl1_ref, wl2_ref, bl2_ref, o_ref):
        C = jnp.sum(p_ref[...], axis=0)
        ii = lax.broadcasted_iota(jnp.int32, (_N, _N), 0)
        jj = lax.broadcasted_iota(jnp.int32, (_N, _N), 1)
        C = C + (ii == jj).astype(jnp.float32)
        deg = jnp.sum(C, axis=0, keepdims=True)
        dis = lax.rsqrt(deg)
        Dm = (ii == jj).astype(jnp.float32) * dis

        def dot(a, b, dims):
            return lax.dot_general(a, b, (dims, ((), ())),
                                   preferred_element_type=jnp.float32)

        A = dot(Dm, dot(C, Dm, (((0,), (0,)))), (((1,), (0,))))

        h = dot(x_ref[...], w1_ref[...], (((1,), (1,))))
        h = jnp.maximum(dot(A, h, (((1,), (0,)))) + b1_ref[...], 0.0)
        h = dot(h, w2_ref[...], (((1,), (1,))))
        h = jnp.maximum(dot(A, h, (((1,), (0,)))) + b2_ref[...], 0.0)
        h = dot(h, w3_ref[...], (((1,), (1,))))
        h = jnp.maximum(dot(A, h, (((1,), (0,)))) + b3_ref[...], 0.0)

        lin1 = bl1_ref[...]
        for n in range(_N):
            lin1 = lin1 + dot(h[n:n + 1, :],
                              wl1_ref[n * 16:(n + 1) * 16, :],
                              (((1,), (0,))))
        q = jnp.maximum(lin1, 0.0)
        o_ref[...] = dot(q, wl2_ref[...], (((1,), (1,)))) + bl2_ref[...]

    return pl.pallas_call(
        body,
        out_shape=jax.ShapeDtypeStruct((1, 16), jnp.float32),
    )(parts, x, W1, b1, W2, b2, W3, b3, Wl1t, bl1, Wl2, bl2)


def kernel(x, edge_index, W_c1, b_c1, W_c2, b_c2, W_c3, b_c3,
           W_l1, b_l1, W_l2, b_l2):
    parts = _sc_edge_histogram(edge_index)
    return _tc_head(
        parts.reshape(_NW, _N, _N), x,
        W_c1, b_c1.reshape(1, -1),
        W_c2, b_c2.reshape(1, -1),
        W_c3, b_c3.reshape(1, -1),
        W_l1.T, b_l1.reshape(1, -1),
        W_l2, b_l2.reshape(1, -1),
    )


# speedup vs baseline: 630.9665x; 1.0007x over previous
import functools

import jax
import jax.numpy as jnp
from jax import lax
from jax.experimental import pallas as pl
from jax.experimental.pallas import tpu as pltpu
from jax.experimental.pallas import tpu_sc as plsc

_N = 8
_E = 1048576
_NC = 2
_NS = 16
_NW = _NC * _NS
_L = 16
_CH = _E // _NW
_NBINS = _N * _N
_UNROLL = 8
_NP = 4
_P = _CH // _NP


def _sc_edge_histogram(edge_index_2d):
    mesh = plsc.VectorSubcoreMesh(core_axis_name="c", subcore_axis_name="s")

    @functools.partial(
        pl.kernel,
        out_type=jax.ShapeDtypeStruct((_NW * _NBINS,), jnp.float32),
        mesh=mesh,
        scratch_types=[
            pltpu.VMEM((2, _P), jnp.int32),
            pltpu.VMEM((2, _P), jnp.int32),
            pltpu.VMEM((_L * _NBINS,), jnp.float32),
            pltpu.VMEM((_NBINS,), jnp.float32),
            pltpu.SemaphoreType.DMA,
            pltpu.SemaphoreType.DMA,
            pltpu.SemaphoreType.DMA,
            pltpu.SemaphoreType.DMA,
        ],
        compiler_params=pltpu.CompilerParams(needs_layout_passes=False),
    )
    def hist(edge_hbm, out_hbm, rows_v, cols_v, acc_v, cnt_v,
             sr0, sr1, sc0, sc1):
        wid = lax.axis_index("s") * _NC + lax.axis_index("c")
        base = wid * _CH
        sems_r = (sr0, sr1)
        sems_c = (sc0, sc1)

        def start(p):
            slot = p % 2
            off = base + p * _P
            return (
                pltpu.async_copy(edge_hbm.at[0, pl.ds(off, _P)],
                                 rows_v.at[slot], sems_r[slot]),
                pltpu.async_copy(edge_hbm.at[1, pl.ds(off, _P)],
                                 cols_v.at[slot], sems_c[slot]),
            )

        inflight = start(0)

        zeros = jnp.zeros((_L,), jnp.float32)
        for j in range(_NBINS):
            acc_v[pl.ds(j * _L, _L)] = zeros

        lane_base = lax.iota(jnp.int32, _L) * _NBINS
        ones = jnp.ones((_L,), jnp.float32)

        for p in range(_NP):
            cur = inflight
            if p + 1 < _NP:
                inflight = start(p + 1)
            cur[0].wait()
            cur[1].wait()
            slot = p % 2

            def body(i, carry):
                base_i = i * (_L * _UNROLL)
                for u in range(_UNROLL):
                    r = rows_v[slot, pl.ds(base_i + u * _L, _L)]
                    c = cols_v[slot, pl.ds(base_i + u * _L, _L)]
                    addr = lane_base + r * _N + c
                    plsc.addupdate_scatter(acc_v, [addr], ones)
                return carry

            lax.fori_loop(0, _P // (_L * _UNROLL), body, 0)

        for kk in range(_NBINS // _L):
            s = acc_v[pl.ds(kk * _L, _L)]
            for l in range(1, _L):
                s = s + acc_v[pl.ds(l * _NBINS + kk * _L, _L)]
            cnt_v[pl.ds(kk * _L, _L)] = s

        pltpu.sync_copy(cnt_v, out_hbm.at[pl.ds(wid * _NBINS, _NBINS)])

    return hist(edge_index_2d)


def _tc_head(parts, x, W1, b1, W2, b2, W3, b3, Wl1t, bl1, Wl2, bl2):
    def body(p_ref, x_ref, w1_ref, b1_ref, w2_ref, b2_ref, w3_ref, b3_ref,
             wl1_ref, bl1_ref, wl2_ref, bl2_ref, o_ref):
        C = jnp.sum(p_ref[...], axis=0)
        ii = lax.broadcasted_iota(jnp.int32, (_N, _N), 0)
        jj = lax.broadcasted_iota(jnp.int32, (_N, _N), 1)
        C = C + (ii == jj).astype(jnp.float32)
        deg = jnp.sum(C, axis=0, keepdims=True)
        dis = lax.rsqrt(deg)
        Dm = (ii == jj).astype(jnp.float32) * dis

        def dot(a, b, dims):
            return lax.dot_general(a, b, (dims, ((), ())),
                                   preferred_element_type=jnp.float32)

        A = dot(Dm, dot(C, Dm, (((0,), (0,)))), (((1,), (0,))))

        h = dot(x_ref[...], w1_ref[...], (((1,), (1,))))
        h = jnp.maximum(dot(A, h, (((1,), (0,)))) + b1_ref[...], 0.0)
        h = dot(h, w2_ref[...], (((1,), (1,))))
        h = jnp.maximum(dot(A, h, (((1,), (0,)))) + b2_ref[...], 0.0)
        h = dot(h, w3_ref[...], (((1,), (1,))))
        h = jnp.maximum(dot(A, h, (((1,), (0,)))) + b3_ref[...], 0.0)

        lin1 = bl1_ref[...]
        for n in range(_N):
            lin1 = lin1 + dot(h[n:n + 1, :],
                              wl1_ref[n * 16:(n + 1) * 16, :],
                              (((1,), (0,))))
        q = jnp.maximum(lin1, 0.0)
        o_ref[...] = dot(q, wl2_ref[...], (((1,), (1,)))) + bl2_ref[...]

    return pl.pallas_call(
        body,
        out_shape=jax.ShapeDtypeStruct((1, 16), jnp.float32),
    )(parts, x, W1, b1, W2, b2, W3, b3, Wl1t, bl1, Wl2, bl2)


def kernel(x, edge_index, W_c1, b_c1, W_c2, b_c2, W_c3, b_c3,
           W_l1, b_l1, W_l2, b_l2):
    parts = _sc_edge_histogram(edge_index)
    return _tc_head(
        parts.reshape(_NW, _N, _N), x,
        W_c1, b_c1.reshape(1, -1),
        W_c2, b_c2.reshape(1, -1),
        W_c3, b_c3.reshape(1, -1),
        W_l1.T, b_l1.reshape(1, -1),
        W_l2, b_l2.reshape(1, -1),
    )
